# baseline (speedup 1.0000x reference)
import functools

import jax
import jax.numpy as jnp
from jax import lax
from jax.experimental import pallas as pl
from jax.experimental.pallas import tpu as pltpu
from jax.experimental.pallas import tpu_sc as plsc

N = 10000
NP = 10240
E = 320000
ECAT = 2 * E
CHUNK = 128
NCHUNK = -(-ECAT // (16 * CHUNK))
NCHUNK += NCHUNK % 2
EPT = NCHUNK * CHUNK
ECATP = 16 * EPT
NCTOT = ECATP // CHUNK
ACC_ROWS = 2 * NP
STRIPE = ACC_ROWS // 16
B_IDX = 100000
BPT = ((B_IDX + 32 * CHUNK - 1) // (32 * CHUNK)) * CHUNK
BPAD = 32 * BPT
NBCHUNK = BPT // CHUNK
BLK = 512
NBLK = NP // BLK
F = 64

_mesh = plsc.VectorSubcoreMesh(core_axis_name="c", subcore_axis_name="s")
_sc_params = pltpu.CompilerParams(use_tc_tiling_on_sc=False,
                                  needs_layout_passes=False)


def _make_passes_kernel(num_tables, passes_per_sc):
    num_passes = 2 * passes_per_sc

    @functools.partial(
        pl.kernel,
        out_type=jax.ShapeDtypeStruct((num_passes, ACC_ROWS, F), jnp.float32),
        mesh=_mesh,
        compiler_params=_sc_params,
        scratch_types=[
            pltpu.VMEM((2, 4, CHUNK), jnp.int32),
            pltpu.VMEM((2, CHUNK), jnp.int32),
            pltpu.VMEM((2, CHUNK), jnp.int32),
            pltpu.VMEM((2, CHUNK, F), jnp.float32),
            pltpu.VMEM((2, CHUNK, F), jnp.float32),
            pltpu.VMEM_SHARED((ACC_ROWS, F), jnp.float32),
            pltpu.SemaphoreType.DMA,
            pltpu.SemaphoreType.DMA,
            pltpu.SemaphoreType.DMA,
        ],
    )
    def passes_kernel(tables, edat, zeros_h, out,
                      IV, IDX, RIDX, G, S, acc, sem_in, sem_g, sem_s):
        c = lax.axis_index("c")
        s = lax.axis_index("s")
        for slot in range(passes_per_sc):
            p = c * passes_per_sc + slot
            vsel = slot % 2
            toff = (p // 2) * NP
            off16 = jnp.full((16,), toff, dtype=jnp.int32)
            cbase = s * NCHUNK

            pltpu.sync_copy(zeros_h, acc.at[pl.ds(s * STRIPE, STRIPE)])
            plsc.subcore_barrier()

            def build_idx(b, _off16=off16):
                for j in range(CHUNK // 16):
                    IDX[b, pl.ds(j * 16, 16)] = (IV[b, 1, pl.ds(j * 16, 16)]
                                                 + _off16)

            pltpu.sync_copy(edat.at[cbase], IV.at[0])
            build_idx(0)
            pltpu.async_copy(tables.at[IDX.at[0]], G.at[0], sem_g)

            def chunk_pair(t2, carry, _vsel=vsel, _cbase=cbase,
                           _build_idx=build_idx):
                for b in range(2):
                    t = t2 * 2 + b
                    q = 1 - b
                    pltpu.async_copy(edat.at[_cbase + t + 1], IV.at[q],
                                     sem_in)
                    pltpu.make_async_copy(tables.at[IDX.at[b]], G.at[b],
                                          sem_g).wait()

                    def edge_body(g, carry2, _b=b):
                        a16 = plsc.bitcast(
                            IV[_b, 2 + _vsel, pl.ds(g * 16, 16)], jnp.float32)
                        RIDX[_b, pl.ds(g * 16, 16)] = IV[_b, 0,
                                                         pl.ds(g * 16, 16)]
                        for u in range(16):
                            e = g * 16 + u
                            a = a16[u]
                            for j in range(F // 16):
                                S[_b, e, pl.ds(j * 16, 16)] = (
                                    a * G[_b, e, pl.ds(j * 16, 16)])
                        return carry2

                    lax.fori_loop(0, CHUNK // 16, edge_body, 0)
                    pltpu.sync_copy(S.at[b], acc.at[RIDX.at[b]], add=True)
                    pltpu.make_async_copy(edat.at[_cbase + t + 1], IV.at[q],
                                          sem_in).wait()
                    _build_idx(q)
                    pltpu.async_copy(tables.at[IDX.at[q]], G.at[q], sem_g)
                return carry

            lax.fori_loop(0, NCHUNK // 2, chunk_pair, 0)
            pltpu.make_async_copy(tables.at[IDX.at[0]], G.at[0], sem_g).wait()
            plsc.subcore_barrier()
            pltpu.sync_copy(acc.at[pl.ds(s * STRIPE, STRIPE)],
                            out.at[p, pl.ds(s * STRIPE, STRIPE)])

    return passes_kernel


_passes8 = _make_passes_kernel(4, 4)
_passes4 = _make_passes_kernel(2, 2)


@functools.partial(
    pl.kernel,
    out_type=jax.ShapeDtypeStruct((BPAD, F), jnp.float32),
    mesh=_mesh,
    compiler_params=_sc_params,
    scratch_types=[
        pltpu.VMEM((CHUNK,), jnp.int32),
        pltpu.VMEM((CHUNK,), jnp.int32),
        pltpu.VMEM((CHUNK, F), jnp.float32),
        pltpu.VMEM((CHUNK, F), jnp.float32),
        pltpu.SemaphoreType.DMA,
        pltpu.SemaphoreType.DMA,
    ],
)
def _pairs_sc(T, i0, i1, out, ia, ib, A, B, sa, sb):
    c = lax.axis_index("c")
    s = lax.axis_index("s")
    wid = c * 16 + s
    base0 = wid * BPT

    def chunk_body(t, carry):
        base = base0 + t * CHUNK
        pltpu.sync_copy(i0.at[pl.ds(base, CHUNK)], ia)
        pltpu.sync_copy(i1.at[pl.ds(base, CHUNK)], ib)
        cpA = pltpu.async_copy(T.at[ia], A, sa)
        cpB = pltpu.async_copy(T.at[ib], B, sb)
        cpA.wait()
        cpB.wait()

        def row_body(e, carry2):
            for j in range(F // 16):
                A[e, pl.ds(j * 16, 16)] = (A[e, pl.ds(j * 16, 16)]
                                           + B[e, pl.ds(j * 16, 16)])
            return carry2

        lax.fori_loop(0, CHUNK, row_body, 0)
        pltpu.sync_copy(A, out.at[pl.ds(base, CHUNK)])
        return carry

    lax.fori_loop(0, NBCHUNK, chunk_body, 0)


def _stage_l1_body(pa_ref, pb_ref, w_ref, b_ref, out_ref):
    def sums(p_ref):
        Sa = jnp.concatenate([p_ref[0], p_ref[2]], axis=1)
        Sc = jnp.concatenate([p_ref[1], p_ref[3]], axis=1)
        Sd = jnp.concatenate([p_ref[4], p_ref[6]], axis=1)
        Sb = jnp.concatenate([p_ref[5], p_ref[7]], axis=1)
        return Sa, Sb, Sc, Sd

    f32 = jnp.float32
    SaA, SbA, ScA, SdA = sums(pa_ref)
    SaB, SbB, ScB, SdB = sums(pb_ref)
    w0 = w_ref[0]
    w1 = w_ref[1]
    dr0 = (jnp.dot(SaA, w0, preferred_element_type=f32)
           - jnp.dot(SbA, w0, preferred_element_type=f32))
    di0 = (jnp.dot(ScA, w0, preferred_element_type=f32)
           + jnp.dot(SdA, w0, preferred_element_type=f32))
    dr1 = (jnp.dot(SaB, w1, preferred_element_type=f32)
           - jnp.dot(SbB, w1, preferred_element_type=f32))
    di1 = (jnp.dot(ScB, w1, preferred_element_type=f32)
           + jnp.dot(SdB, w1, preferred_element_type=f32))
    r = (dr0 + dr1) + b_ref[0]
    i = (di0 + di1) + b_ref[0]
    m = (r >= 0).astype(f32)
    out_ref[0] = r * m
    out_ref[1] = i * m


def _stage_l1(sums1, W1, b1):
    return pl.pallas_call(
        _stage_l1_body,
        grid=(NBLK,),
        in_specs=[
            pl.BlockSpec((8, BLK, F), lambda r: (0, r, 0)),
            pl.BlockSpec((8, BLK, F), lambda r: (0, NBLK + r, 0)),
            pl.BlockSpec((2, 128, F), lambda r: (0, 0, 0)),
            pl.BlockSpec((1, F), lambda r: (0, 0)),
        ],
        out_specs=pl.BlockSpec((2, BLK, F), lambda r: (0, r, 0)),
        out_shape=jax.ShapeDtypeStruct((2, NP, F), jnp.float32),
    )(sums1, sums1, W1, b1)


def _stage_l2_body(pa_ref, pb_ref, w_ref, b_ref, wl_ref, bl_ref, out_ref):
    f32 = jnp.float32
    w0 = w_ref[0]
    w1 = w_ref[1]
    dr0 = (jnp.dot(pa_ref[0], w0, preferred_element_type=f32)
           - jnp.dot(pa_ref[3], w0, preferred_element_type=f32))
    di0 = (jnp.dot(pa_ref[1], w0, preferred_element_type=f32)
           + jnp.dot(pa_ref[2], w0, preferred_element_type=f32))
    dr1 = (jnp.dot(pb_ref[0], w1, preferred_element_type=f32)
           - jnp.dot(pb_ref[3], w1, preferred_element_type=f32))
    di1 = (jnp.dot(pb_ref[1], w1, preferred_element_type=f32)
           + jnp.dot(pb_ref[2], w1, preferred_element_type=f32))
    r = (dr0 + dr1) + b_ref[0]
    i = (di0 + di1) + b_ref[0]
    m = (r >= 0).astype(f32)
    r = r * m
    i = i * m
    U = (jnp.dot(r, wl_ref[0], preferred_element_type=f32)
         + jnp.dot(i, wl_ref[2], preferred_element_type=f32)
         + bl_ref[0])
    V = (jnp.dot(r, wl_ref[1], preferred_element_type=f32)
         + jnp.dot(i, wl_ref[3], preferred_element_type=f32))
    out_ref[0] = U
    out_ref[1] = V


def _stage_l2(sums2, W2, b2, wl, blin2d):
    return pl.pallas_call(
        _stage_l2_body,
        grid=(NBLK,),
        in_specs=[
            pl.BlockSpec((4, BLK, F), lambda r: (0, r, 0)),
            pl.BlockSpec((4, BLK, F), lambda r: (0, NBLK + r, 0)),
            pl.BlockSpec((2, F, F), lambda r: (0, 0, 0)),
            pl.BlockSpec((1, F), lambda r: (0, 0)),
            pl.BlockSpec((4, F, F), lambda r: (0, 0, 0)),
            pl.BlockSpec((1, F), lambda r: (0, 0)),
        ],
        out_specs=pl.BlockSpec((2, BLK, F), lambda r: (0, r, 0)),
        out_shape=jax.ShapeDtypeStruct((2, NP, F), jnp.float32),
    )(sums2, sums2, W2, b2, wl, blin2d)


def kernel(real, imag, rows0, cols0, vr0, vi0, rows1, cols1, vr1, vi1,
           W1, b1, W2, b2, Wlin, blin, index):
    f32 = jnp.float32
    realp = jnp.pad(real, ((0, NP - N), (0, 0)))
    imagp = jnp.pad(imag, ((0, NP - N), (0, 0)))

    pad_e = ECATP - ECAT
    rows2 = jnp.pad(jnp.concatenate([rows0, rows1 + NP]), (0, pad_e))
    cols_cat = jnp.pad(jnp.concatenate([cols0, cols1]), (0, pad_e))
    vr_b = lax.bitcast_convert_type(
        jnp.pad(jnp.concatenate([vr0, vr1]), (0, pad_e)), jnp.int32)
    vi_b = lax.bitcast_convert_type(
        jnp.pad(jnp.concatenate([vi0, vi1]), (0, pad_e)), jnp.int32)
    edat = jnp.stack([rows2, cols_cat, vr_b, vi_b])
    edat = edat.reshape(4, NCTOT, CHUNK).transpose(1, 0, 2)
    edat = jnp.pad(edat, ((0, 1), (0, 0), (0, 0)))

    zeros_h = jnp.zeros((STRIPE, F), dtype=f32)

    tables1 = jnp.concatenate([realp[:, :F], realp[:, F:],
                               imagp[:, :F], imagp[:, F:]], axis=0)
    sums1 = _passes8(tables1, edat, zeros_h)
    tbl2 = _stage_l1(sums1, W1, b1)

    tables2 = tbl2.reshape(2 * NP, F)
    sums2 = _passes4(tables2, edat, zeros_h)
    uv = _stage_l2(sums2, W2, b2, Wlin.reshape(4, F, F), blin.reshape(1, F))

    T = uv.reshape(2 * NP, F)
    i0 = jnp.pad(index[:, 0], (0, BPAD - B_IDX))
    i1 = jnp.pad(index[:, 1], (0, BPAD - B_IDX)) + NP
    out = _pairs_sc(T, i0, i1)
    return out[:B_IDX]


# speedup vs baseline: 1.9250x; 1.9250x over previous
import functools

import jax
import jax.numpy as jnp
from jax import lax
from jax.experimental import pallas as pl
from jax.experimental.pallas import tpu as pltpu
from jax.experimental.pallas import tpu_sc as plsc

N = 10000
NP = 10240
E = 320000
ECAT = 2 * E
CHUNK = 128
NCHUNK = -(-ECAT // (16 * CHUNK))
NCHUNK += NCHUNK % 2
EPT = NCHUNK * CHUNK
ECATP = 16 * EPT
NCTOT = ECATP // CHUNK
ACC_ROWS = 2 * NP
STRIPE = ACC_ROWS // 16
B_IDX = 100000
BPT = ((B_IDX + 32 * CHUNK - 1) // (32 * CHUNK)) * CHUNK
BPAD = 32 * BPT
NBCHUNK = BPT // CHUNK
BLK = 512
NBLK = NP // BLK
F = 64

_mesh = plsc.VectorSubcoreMesh(core_axis_name="c", subcore_axis_name="s")
_sc_params = pltpu.CompilerParams(use_tc_tiling_on_sc=False,
                                  needs_layout_passes=False)


def _make_passes_kernel(num_tables, passes_per_sc):
    num_passes = 2 * passes_per_sc

    @functools.partial(
        pl.kernel,
        out_type=jax.ShapeDtypeStruct((num_passes, ACC_ROWS, F), jnp.float32),
        mesh=_mesh,
        compiler_params=_sc_params,
        scratch_types=[
            pltpu.VMEM((2, 4, CHUNK), jnp.int32),
            pltpu.VMEM((2, CHUNK), jnp.int32),
            pltpu.VMEM((2, CHUNK, F), jnp.float32),
            pltpu.VMEM((CHUNK, F), jnp.float32),
            pltpu.VMEM_SHARED((ACC_ROWS, F), jnp.float32),
            pltpu.SemaphoreType.DMA,
            pltpu.SemaphoreType.DMA,
        ],
    )
    def passes_kernel(tables, edat, zeros_h, out,
                      IV, IDX, G, S, acc, sem_in, sem_g):
        c = lax.axis_index("c")
        s = lax.axis_index("s")
        for slot in range(passes_per_sc):
            p = c * passes_per_sc + slot
            vsel = slot % 2
            toff = (p // 2) * NP
            off16 = jnp.full((16,), toff, dtype=jnp.int32)
            cbase = s * NCHUNK

            pltpu.sync_copy(zeros_h, acc.at[pl.ds(s * STRIPE, STRIPE)])
            plsc.subcore_barrier()

            def build_idx(b, _off16=off16):
                for j in range(CHUNK // 16):
                    IDX[b, pl.ds(j * 16, 16)] = (IV[b, 1, pl.ds(j * 16, 16)]
                                                 + _off16)

            pltpu.sync_copy(edat.at[cbase], IV.at[0])
            build_idx(0)
            pltpu.async_copy(tables.at[IDX.at[0]], G.at[0], sem_g)

            def chunk_pair(t2, carry, _vsel=vsel, _cbase=cbase,
                           _build_idx=build_idx):
                for b in range(2):
                    t = t2 * 2 + b
                    q = 1 - b
                    pltpu.async_copy(edat.at[_cbase + t + 1], IV.at[q],
                                     sem_in)
                    pltpu.make_async_copy(tables.at[IDX.at[b]], G.at[b],
                                          sem_g).wait()

                    def edge_body(g, carry2, _b=b):
                        a16 = plsc.bitcast(
                            IV[_b, 2 + _vsel, pl.ds(g * 16, 16)], jnp.float32)
                        for u in range(16):
                            e = g * 16 + u
                            a = a16[u]
                            for j in range(F // 16):
                                S[e, pl.ds(j * 16, 16)] = (
                                    a * G[_b, e, pl.ds(j * 16, 16)])
                        return carry2

                    lax.fori_loop(0, CHUNK // 16, edge_body, 0)
                    pltpu.sync_copy(S, acc.at[IV.at[b, 0]], add=True)
                    pltpu.make_async_copy(edat.at[_cbase + t + 1], IV.at[q],
                                          sem_in).wait()
                    _build_idx(q)
                    pltpu.async_copy(tables.at[IDX.at[q]], G.at[q], sem_g)
                return carry

            lax.fori_loop(0, NCHUNK // 2, chunk_pair, 0)
            pltpu.make_async_copy(tables.at[IDX.at[0]], G.at[0], sem_g).wait()
            plsc.subcore_barrier()
            pltpu.sync_copy(acc.at[pl.ds(s * STRIPE, STRIPE)],
                            out.at[p, pl.ds(s * STRIPE, STRIPE)])

    return passes_kernel


_passes8 = _make_passes_kernel(4, 4)
_passes4 = _make_passes_kernel(2, 2)


@functools.partial(
    pl.kernel,
    out_type=jax.ShapeDtypeStruct((BPAD, F), jnp.float32),
    mesh=_mesh,
    compiler_params=_sc_params,
    scratch_types=[
        pltpu.VMEM((CHUNK,), jnp.int32),
        pltpu.VMEM((CHUNK,), jnp.int32),
        pltpu.VMEM((CHUNK, F), jnp.float32),
        pltpu.VMEM((CHUNK, F), jnp.float32),
        pltpu.SemaphoreType.DMA,
        pltpu.SemaphoreType.DMA,
    ],
)
def _pairs_sc(T, i0, i1, out, ia, ib, A, B, sa, sb):
    c = lax.axis_index("c")
    s = lax.axis_index("s")
    wid = c * 16 + s
    base0 = wid * BPT

    def chunk_body(t, carry):
        base = base0 + t * CHUNK
        pltpu.sync_copy(i0.at[pl.ds(base, CHUNK)], ia)
        pltpu.sync_copy(i1.at[pl.ds(base, CHUNK)], ib)
        cpA = pltpu.async_copy(T.at[ia], A, sa)
        cpB = pltpu.async_copy(T.at[ib], B, sb)
        cpA.wait()
        cpB.wait()

        def row_body(e, carry2):
            for j in range(F // 16):
                A[e, pl.ds(j * 16, 16)] = (A[e, pl.ds(j * 16, 16)]
                                           + B[e, pl.ds(j * 16, 16)])
            return carry2

        lax.fori_loop(0, CHUNK, row_body, 0)
        pltpu.sync_copy(A, out.at[pl.ds(base, CHUNK)])
        return carry

    lax.fori_loop(0, NBCHUNK, chunk_body, 0)


def _stage_l1_body(pa_ref, pb_ref, w_ref, b_ref, out_ref):
    def sums(p_ref):
        Sa = jnp.concatenate([p_ref[0], p_ref[2]], axis=1)
        Sc = jnp.concatenate([p_ref[1], p_ref[3]], axis=1)
        Sd = jnp.concatenate([p_ref[4], p_ref[6]], axis=1)
        Sb = jnp.concatenate([p_ref[5], p_ref[7]], axis=1)
        return Sa, Sb, Sc, Sd

    f32 = jnp.float32
    SaA, SbA, ScA, SdA = sums(pa_ref)
    SaB, SbB, ScB, SdB = sums(pb_ref)
    w0 = w_ref[0]
    w1 = w_ref[1]
    dr0 = (jnp.dot(SaA, w0, preferred_element_type=f32)
           - jnp.dot(SbA, w0, preferred_element_type=f32))
    di0 = (jnp.dot(ScA, w0, preferred_element_type=f32)
           + jnp.dot(SdA, w0, preferred_element_type=f32))
    dr1 = (jnp.dot(SaB, w1, preferred_element_type=f32)
           - jnp.dot(SbB, w1, preferred_element_type=f32))
    di1 = (jnp.dot(ScB, w1, preferred_element_type=f32)
           + jnp.dot(SdB, w1, preferred_element_type=f32))
    r = (dr0 + dr1) + b_ref[0]
    i = (di0 + di1) + b_ref[0]
    m = (r >= 0).astype(f32)
    out_ref[0] = r * m
    out_ref[1] = i * m


def _stage_l1(sums1, W1, b1):
    return pl.pallas_call(
        _stage_l1_body,
        grid=(NBLK,),
        in_specs=[
            pl.BlockSpec((8, BLK, F), lambda r: (0, r, 0)),
            pl.BlockSpec((8, BLK, F), lambda r: (0, NBLK + r, 0)),
            pl.BlockSpec((2, 128, F), lambda r: (0, 0, 0)),
            pl.BlockSpec((1, F), lambda r: (0, 0)),
        ],
        out_specs=pl.BlockSpec((2, BLK, F), lambda r: (0, r, 0)),
        out_shape=jax.ShapeDtypeStruct((2, NP, F), jnp.float32),
    )(sums1, sums1, W1, b1)


def _stage_l2_body(pa_ref, pb_ref, w_ref, b_ref, wl_ref, bl_ref, out_ref):
    f32 = jnp.float32
    w0 = w_ref[0]
    w1 = w_ref[1]
    dr0 = (jnp.dot(pa_ref[0], w0, preferred_element_type=f32)
           - jnp.dot(pa_ref[3], w0, preferred_element_type=f32))
    di0 = (jnp.dot(pa_ref[1], w0, preferred_element_type=f32)
           + jnp.dot(pa_ref[2], w0, preferred_element_type=f32))
    dr1 = (jnp.dot(pb_ref[0], w1, preferred_element_type=f32)
           - jnp.dot(pb_ref[3], w1, preferred_element_type=f32))
    di1 = (jnp.dot(pb_ref[1], w1, preferred_element_type=f32)
           + jnp.dot(pb_ref[2], w1, preferred_element_type=f32))
    r = (dr0 + dr1) + b_ref[0]
    i = (di0 + di1) + b_ref[0]
    m = (r >= 0).astype(f32)
    r = r * m
    i = i * m
    U = (jnp.dot(r, wl_ref[0], preferred_element_type=f32)
         + jnp.dot(i, wl_ref[2], preferred_element_type=f32)
         + bl_ref[0])
    V = (jnp.dot(r, wl_ref[1], preferred_element_type=f32)
         + jnp.dot(i, wl_ref[3], preferred_element_type=f32))
    out_ref[0] = U
    out_ref[1] = V


def _stage_l2(sums2, W2, b2, wl, blin2d):
    return pl.pallas_call(
        _stage_l2_body,
        grid=(NBLK,),
        in_specs=[
            pl.BlockSpec((4, BLK, F), lambda r: (0, r, 0)),
            pl.BlockSpec((4, BLK, F), lambda r: (0, NBLK + r, 0)),
            pl.BlockSpec((2, F, F), lambda r: (0, 0, 0)),
            pl.BlockSpec((1, F), lambda r: (0, 0)),
            pl.BlockSpec((4, F, F), lambda r: (0, 0, 0)),
            pl.BlockSpec((1, F), lambda r: (0, 0)),
        ],
        out_specs=pl.BlockSpec((2, BLK, F), lambda r: (0, r, 0)),
        out_shape=jax.ShapeDtypeStruct((2, NP, F), jnp.float32),
    )(sums2, sums2, W2, b2, wl, blin2d)


def kernel(real, imag, rows0, cols0, vr0, vi0, rows1, cols1, vr1, vi1,
           W1, b1, W2, b2, Wlin, blin, index):
    f32 = jnp.float32
    realp = jnp.pad(real, ((0, NP - N), (0, 0)))
    imagp = jnp.pad(imag, ((0, NP - N), (0, 0)))

    pad_e = ECATP - ECAT
    rows2 = jnp.pad(jnp.concatenate([rows0, rows1 + NP]), (0, pad_e))
    cols_cat = jnp.pad(jnp.concatenate([cols0, cols1]), (0, pad_e))
    vr_b = lax.bitcast_convert_type(
        jnp.pad(jnp.concatenate([vr0, vr1]), (0, pad_e)), jnp.int32)
    vi_b = lax.bitcast_convert_type(
        jnp.pad(jnp.concatenate([vi0, vi1]), (0, pad_e)), jnp.int32)
    edat = jnp.stack([rows2, cols_cat, vr_b, vi_b])
    edat = edat.reshape(4, NCTOT, CHUNK).transpose(1, 0, 2)
    edat = jnp.pad(edat, ((0, 1), (0, 0), (0, 0)))

    zeros_h = jnp.zeros((STRIPE, F), dtype=f32)

    tables1 = jnp.concatenate([realp[:, :F], realp[:, F:],
                               imagp[:, :F], imagp[:, F:]], axis=0)
    sums1 = _passes8(tables1, edat, zeros_h)
    tbl2 = _stage_l1(sums1, W1, b1)

    tables2 = tbl2.reshape(2 * NP, F)
    sums2 = _passes4(tables2, edat, zeros_h)
    uv = _stage_l2(sums2, W2, b2, Wlin.reshape(4, F, F), blin.reshape(1, F))

    T = uv.reshape(2 * NP, F)
    i0 = jnp.pad(index[:, 0], (0, BPAD - B_IDX))
    i1 = jnp.pad(index[:, 1], (0, BPAD - B_IDX)) + NP
    out = _pairs_sc(T, i0, i1)
    return out[:B_IDX]


# speedup vs baseline: 2.3330x; 1.2119x over previous
import functools

import jax
import jax.numpy as jnp
from jax import lax
from jax.experimental import pallas as pl
from jax.experimental.pallas import tpu as pltpu
from jax.experimental.pallas import tpu_sc as plsc

N = 10000
NP = 10240
E = 320000
ECAT = 2 * E
CHUNK = 128
NCHUNK = -(-ECAT // (16 * CHUNK))
NCHUNK += (-NCHUNK) % 4
EPT = NCHUNK * CHUNK
ECATP = 16 * EPT
NCTOT = ECATP // CHUNK
ACC_ROWS = 2 * NP
STRIPE = ACC_ROWS // 16
B_IDX = 100000
BPT = ((B_IDX + 32 * CHUNK - 1) // (32 * CHUNK)) * CHUNK
BPAD = 32 * BPT
NBCHUNK = BPT // CHUNK
BLK = 512
NBLK = NP // BLK
F = 64

_mesh = plsc.VectorSubcoreMesh(core_axis_name="c", subcore_axis_name="s")
_sc_params = pltpu.CompilerParams(use_tc_tiling_on_sc=False,
                                  needs_layout_passes=False)


def _make_passes_kernel(num_tables, passes_per_sc):
    num_passes = 2 * passes_per_sc

    @functools.partial(
        pl.kernel,
        out_type=jax.ShapeDtypeStruct((num_passes, ACC_ROWS, F), jnp.float32),
        mesh=_mesh,
        compiler_params=_sc_params,
        scratch_types=[
            pltpu.VMEM((4, 4, CHUNK), jnp.int32),
            pltpu.VMEM((2, CHUNK), jnp.int32),
            pltpu.VMEM((2, CHUNK, F), jnp.float32),
            pltpu.VMEM((CHUNK, F), jnp.float32),
            pltpu.VMEM_SHARED((ACC_ROWS, F), jnp.float32),
            pltpu.SemaphoreType.DMA,
            pltpu.SemaphoreType.DMA,
        ],
    )
    def passes_kernel(tables, edat, zeros_h, out,
                      IV, IDX, G, S, acc, sem_in, sem_g):
        c = lax.axis_index("c")
        s = lax.axis_index("s")
        for slot in range(passes_per_sc):
            p = c * passes_per_sc + slot
            vsel = slot % 2
            toff = (p // 2) * NP
            off16 = jnp.full((16,), toff, dtype=jnp.int32)
            cbase = s * NCHUNK

            pltpu.sync_copy(zeros_h, acc.at[pl.ds(s * STRIPE, STRIPE)])
            plsc.subcore_barrier()

            def build_idx(b, d, _off16=off16):
                for j in range(CHUNK // 16):
                    IDX[d, pl.ds(j * 16, 16)] = (IV[b, 1, pl.ds(j * 16, 16)]
                                                 + _off16)

            pltpu.sync_copy(edat.at[cbase], IV.at[0])
            build_idx(0, 0)
            pltpu.async_copy(tables.at[IDX.at[0]], G.at[0], sem_g)
            pltpu.async_copy(edat.at[cbase + 1], IV.at[1], sem_in)

            def chunk_quad(t4, carry, _vsel=vsel, _cbase=cbase,
                           _build_idx=build_idx):
                for b in range(4):
                    t = t4 * 4 + b
                    gb = b % 2
                    gq = 1 - gb
                    nb = (b + 1) % 4
                    pltpu.make_async_copy(tables.at[IDX.at[gb]], G.at[gb],
                                          sem_g).wait()
                    pltpu.make_async_copy(edat.at[_cbase + t + 1], IV.at[nb],
                                          sem_in).wait()
                    _build_idx(nb, gq)
                    pltpu.async_copy(tables.at[IDX.at[gq]], G.at[gq], sem_g)
                    pltpu.async_copy(edat.at[_cbase + t + 2],
                                     IV.at[(b + 2) % 4], sem_in)

                    def edge_body(g, carry2, _b=b, _gb=gb):
                        a16 = plsc.bitcast(
                            IV[_b, 2 + _vsel, pl.ds(g * 16, 16)], jnp.float32)
                        for u in range(16):
                            e = g * 16 + u
                            a = a16[u]
                            for j in range(F // 16):
                                S[e, pl.ds(j * 16, 16)] = (
                                    a * G[_gb, e, pl.ds(j * 16, 16)])
                        return carry2

                    lax.fori_loop(0, CHUNK // 16, edge_body, 0)
                    pltpu.sync_copy(S, acc.at[IV.at[b, 0]], add=True)
                return carry

            lax.fori_loop(0, NCHUNK // 4, chunk_quad, 0)
            pltpu.make_async_copy(tables.at[IDX.at[0]], G.at[0], sem_g).wait()
            pltpu.make_async_copy(edat.at[cbase + NCHUNK + 1], IV.at[1],
                                  sem_in).wait()
            plsc.subcore_barrier()
            pltpu.sync_copy(acc.at[pl.ds(s * STRIPE, STRIPE)],
                            out.at[p, pl.ds(s * STRIPE, STRIPE)])

    return passes_kernel


_passes8 = _make_passes_kernel(4, 4)
_passes4 = _make_passes_kernel(2, 2)


@functools.partial(
    pl.kernel,
    out_type=jax.ShapeDtypeStruct((BPAD, F), jnp.float32),
    mesh=_mesh,
    compiler_params=_sc_params,
    scratch_types=[
        pltpu.VMEM((CHUNK,), jnp.int32),
        pltpu.VMEM((CHUNK,), jnp.int32),
        pltpu.VMEM((CHUNK, F), jnp.float32),
        pltpu.VMEM((CHUNK, F), jnp.float32),
        pltpu.SemaphoreType.DMA,
        pltpu.SemaphoreType.DMA,
    ],
)
def _pairs_sc(T, i0, i1, out, ia, ib, A, B, sa, sb):
    c = lax.axis_index("c")
    s = lax.axis_index("s")
    wid = c * 16 + s
    base0 = wid * BPT

    def chunk_body(t, carry):
        base = base0 + t * CHUNK
        pltpu.sync_copy(i0.at[pl.ds(base, CHUNK)], ia)
        pltpu.sync_copy(i1.at[pl.ds(base, CHUNK)], ib)
        cpA = pltpu.async_copy(T.at[ia], A, sa)
        cpB = pltpu.async_copy(T.at[ib], B, sb)
        cpA.wait()
        cpB.wait()

        def row_body(e, carry2):
            for j in range(F // 16):
                A[e, pl.ds(j * 16, 16)] = (A[e, pl.ds(j * 16, 16)]
                                           + B[e, pl.ds(j * 16, 16)])
            return carry2

        lax.fori_loop(0, CHUNK, row_body, 0)
        pltpu.sync_copy(A, out.at[pl.ds(base, CHUNK)])
        return carry

    lax.fori_loop(0, NBCHUNK, chunk_body, 0)


def _stage_l1_body(pa_ref, pb_ref, w_ref, b_ref, out_ref):
    def sums(p_ref):
        Sa = jnp.concatenate([p_ref[0], p_ref[2]], axis=1)
        Sc = jnp.concatenate([p_ref[1], p_ref[3]], axis=1)
        Sd = jnp.concatenate([p_ref[4], p_ref[6]], axis=1)
        Sb = jnp.concatenate([p_ref[5], p_ref[7]], axis=1)
        return Sa, Sb, Sc, Sd

    f32 = jnp.float32
    SaA, SbA, ScA, SdA = sums(pa_ref)
    SaB, SbB, ScB, SdB = sums(pb_ref)
    w0 = w_ref[0]
    w1 = w_ref[1]
    dr0 = (jnp.dot(SaA, w0, preferred_element_type=f32)
           - jnp.dot(SbA, w0, preferred_element_type=f32))
    di0 = (jnp.dot(ScA, w0, preferred_element_type=f32)
           + jnp.dot(SdA, w0, preferred_element_type=f32))
    dr1 = (jnp.dot(SaB, w1, preferred_element_type=f32)
           - jnp.dot(SbB, w1, preferred_element_type=f32))
    di1 = (jnp.dot(ScB, w1, preferred_element_type=f32)
           + jnp.dot(SdB, w1, preferred_element_type=f32))
    r = (dr0 + dr1) + b_ref[0]
    i = (di0 + di1) + b_ref[0]
    m = (r >= 0).astype(f32)
    out_ref[0] = r * m
    out_ref[1] = i * m


def _stage_l1(sums1, W1, b1):
    return pl.pallas_call(
        _stage_l1_body,
        grid=(NBLK,),
        in_specs=[
            pl.BlockSpec((8, BLK, F), lambda r: (0, r, 0)),
            pl.BlockSpec((8, BLK, F), lambda r: (0, NBLK + r, 0)),
            pl.BlockSpec((2, 128, F), lambda r: (0, 0, 0)),
            pl.BlockSpec((1, F), lambda r: (0, 0)),
        ],
        out_specs=pl.BlockSpec((2, BLK, F), lambda r: (0, r, 0)),
        out_shape=jax.ShapeDtypeStruct((2, NP, F), jnp.float32),
    )(sums1, sums1, W1, b1)


def _stage_l2_body(pa_ref, pb_ref, w_ref, b_ref, wl_ref, bl_ref, out_ref):
    f32 = jnp.float32
    w0 = w_ref[0]
    w1 = w_ref[1]
    dr0 = (jnp.dot(pa_ref[0], w0, preferred_element_type=f32)
           - jnp.dot(pa_ref[3], w0, preferred_element_type=f32))
    di0 = (jnp.dot(pa_ref[1], w0, preferred_element_type=f32)
           + jnp.dot(pa_ref[2], w0, preferred_element_type=f32))
    dr1 = (jnp.dot(pb_ref[0], w1, preferred_element_type=f32)
           - jnp.dot(pb_ref[3], w1, preferred_element_type=f32))
    di1 = (jnp.dot(pb_ref[1], w1, preferred_element_type=f32)
           + jnp.dot(pb_ref[2], w1, preferred_element_type=f32))
    r = (dr0 + dr1) + b_ref[0]
    i = (di0 + di1) + b_ref[0]
    m = (r >= 0).astype(f32)
    r = r * m
    i = i * m
    U = (jnp.dot(r, wl_ref[0], preferred_element_type=f32)
         + jnp.dot(i, wl_ref[2], preferred_element_type=f32)
         + bl_ref[0])
    V = (jnp.dot(r, wl_ref[1], preferred_element_type=f32)
         + jnp.dot(i, wl_ref[3], preferred_element_type=f32))
    out_ref[0] = U
    out_ref[1] = V


def _stage_l2(sums2, W2, b2, wl, blin2d):
    return pl.pallas_call(
        _stage_l2_body,
        grid=(NBLK,),
        in_specs=[
            pl.BlockSpec((4, BLK, F), lambda r: (0, r, 0)),
            pl.BlockSpec((4, BLK, F), lambda r: (0, NBLK + r, 0)),
            pl.BlockSpec((2, F, F), lambda r: (0, 0, 0)),
            pl.BlockSpec((1, F), lambda r: (0, 0)),
            pl.BlockSpec((4, F, F), lambda r: (0, 0, 0)),
            pl.BlockSpec((1, F), lambda r: (0, 0)),
        ],
        out_specs=pl.BlockSpec((2, BLK, F), lambda r: (0, r, 0)),
        out_shape=jax.ShapeDtypeStruct((2, NP, F), jnp.float32),
    )(sums2, sums2, W2, b2, wl, blin2d)


def kernel(real, imag, rows0, cols0, vr0, vi0, rows1, cols1, vr1, vi1,
           W1, b1, W2, b2, Wlin, blin, index):
    f32 = jnp.float32
    realp = jnp.pad(real, ((0, NP - N), (0, 0)))
    imagp = jnp.pad(imag, ((0, NP - N), (0, 0)))

    pad_e = ECATP - ECAT
    rows2 = jnp.pad(jnp.concatenate([rows0, rows1 + NP]), (0, pad_e))
    cols_cat = jnp.pad(jnp.concatenate([cols0, cols1]), (0, pad_e))
    vr_b = lax.bitcast_convert_type(
        jnp.pad(jnp.concatenate([vr0, vr1]), (0, pad_e)), jnp.int32)
    vi_b = lax.bitcast_convert_type(
        jnp.pad(jnp.concatenate([vi0, vi1]), (0, pad_e)), jnp.int32)
    edat = jnp.stack([rows2, cols_cat, vr_b, vi_b])
    edat = edat.reshape(4, NCTOT, CHUNK).transpose(1, 0, 2)
    edat = jnp.pad(edat, ((0, 2), (0, 0), (0, 0)))

    zeros_h = jnp.zeros((STRIPE, F), dtype=f32)

    tables1 = jnp.concatenate([realp[:, :F], realp[:, F:],
                               imagp[:, :F], imagp[:, F:]], axis=0)
    sums1 = _passes8(tables1, edat, zeros_h)
    tbl2 = _stage_l1(sums1, W1, b1)

    tables2 = tbl2.reshape(2 * NP, F)
    sums2 = _passes4(tables2, edat, zeros_h)
    uv = _stage_l2(sums2, W2, b2, Wlin.reshape(4, F, F), blin.reshape(1, F))

    T = uv.reshape(2 * NP, F)
    i0 = jnp.pad(index[:, 0], (0, BPAD - B_IDX))
    i1 = jnp.pad(index[:, 1], (0, BPAD - B_IDX)) + NP
    out = _pairs_sc(T, i0, i1)
    return out[:B_IDX]
